# baseline (speedup 1.0000x reference)
import functools

import jax
import jax.numpy as jnp
from jax import lax
from jax.experimental import pallas as pl
from jax.experimental.pallas import tpu as pltpu, tpu_sc as plsc

MAX_POS = 8192
EMB = 128

_info = plsc.get_sparse_core_info()
_NC, _NS = _info.num_cores, _info.num_subcores
_NW = _NC * _NS

_ROWS, _COLS = 4, 8192
_B = _ROWS * _COLS
_PER_W = _B // _NW
_WPR = _COLS // _PER_W
_GB = 128
_NG = _PER_W // _GB
_K = 5


def _make_kernel():
    mesh = plsc.VectorSubcoreMesh(core_axis_name="c", subcore_axis_name="s")

    @functools.partial(
        pl.kernel,
        mesh=mesh,
        out_type=jax.ShapeDtypeStruct((_B, EMB), jnp.float32),
        scratch_types=[
            pltpu.VMEM((_PER_W,), jnp.int32),
        ]
        + [pltpu.VMEM((_GB, EMB), jnp.float32) for _ in range(_K)]
        + [pltpu.SemaphoreType.DMA for _ in range(2 * _K)],
    )
    def gather_kernel(idx_hbm, table_hbm, out_hbm, idx_v, *bufs_and_sems):
        bufs = bufs_and_sems[:_K]
        gsems = bufs_and_sems[_K : 2 * _K]
        wsems = bufs_and_sems[2 * _K : 3 * _K]
        wid = lax.axis_index("s") * _NC + lax.axis_index("c")
        pltpu.sync_copy(
            idx_hbm.at[wid // _WPR, pl.ds((wid % _WPR) * _PER_W, _PER_W)], idx_v
        )

        def start_gather(j):
            b = j % _K
            return pltpu.async_copy(
                table_hbm.at[idx_v.at[pl.ds(j * _GB, _GB)]], bufs[b], gsems[b]
            )

        def start_write(j):
            b = j % _K
            base = wid * _PER_W + j * _GB
            return pltpu.async_copy(bufs[b], out_hbm.at[pl.ds(base, _GB)], wsems[b])

        gathers = {j: start_gather(j) for j in range(_K)}
        writes = {}
        for j in range(_NG):
            gathers.pop(j).wait()
            if j - 1 in writes and j - 1 + _K < _NG:
                writes.pop(j - 1).wait()
                gathers[j - 1 + _K] = start_gather(j - 1 + _K)
            writes[j] = start_write(j)
        for j in sorted(writes):
            writes.pop(j).wait()

    return gather_kernel


_gather = _make_kernel()


def kernel(position_ids, table):
    out = _gather(position_ids.astype(jnp.int32), table)
    return out.reshape(position_ids.shape + (EMB,))


# speedup vs baseline: 1.0145x; 1.0145x over previous
import functools

import jax
import jax.numpy as jnp
from jax import lax
from jax.experimental import pallas as pl
from jax.experimental.pallas import tpu as pltpu, tpu_sc as plsc

MAX_POS = 8192
EMB = 128

_info = plsc.get_sparse_core_info()
_NC, _NS = _info.num_cores, _info.num_subcores
_NW = _NC * _NS

_ROWS, _COLS = 4, 8192
_B = _ROWS * _COLS
_PER_W = _B // _NW
_WPR = _COLS // _PER_W
_GB = 128
_NG = _PER_W // _GB
_K = 6


def _make_kernel():
    mesh = plsc.VectorSubcoreMesh(core_axis_name="c", subcore_axis_name="s")

    @functools.partial(
        pl.kernel,
        mesh=mesh,
        out_type=jax.ShapeDtypeStruct((_B, EMB), jnp.float32),
        scratch_types=[
            pltpu.VMEM((_PER_W,), jnp.int32),
        ]
        + [pltpu.VMEM((_GB, EMB), jnp.float32) for _ in range(_K)]
        + [pltpu.SemaphoreType.DMA for _ in range(2 * _K)],
    )
    def gather_kernel(idx_hbm, table_hbm, out_hbm, idx_v, *bufs_and_sems):
        bufs = bufs_and_sems[:_K]
        gsems = bufs_and_sems[_K : 2 * _K]
        wsems = bufs_and_sems[2 * _K : 3 * _K]
        wid = lax.axis_index("s") * _NC + lax.axis_index("c")
        pltpu.sync_copy(
            idx_hbm.at[wid // _WPR, pl.ds((wid % _WPR) * _PER_W, _PER_W)], idx_v
        )

        def start_gather(j):
            b = j % _K
            return pltpu.async_copy(
                table_hbm.at[idx_v.at[pl.ds(j * _GB, _GB)]], bufs[b], gsems[b]
            )

        def start_write(j):
            b = j % _K
            base = wid * _PER_W + j * _GB
            return pltpu.async_copy(bufs[b], out_hbm.at[pl.ds(base, _GB)], wsems[b])

        gathers = {j: start_gather(j) for j in range(_K)}
        writes = {}
        for j in range(_NG):
            gathers.pop(j).wait()
            if j - 1 in writes and j - 1 + _K < _NG:
                writes.pop(j - 1).wait()
                gathers[j - 1 + _K] = start_gather(j - 1 + _K)
            writes[j] = start_write(j)
        for j in sorted(writes):
            writes.pop(j).wait()

    return gather_kernel


_gather = _make_kernel()


def kernel(position_ids, table):
    out = _gather(position_ids.astype(jnp.int32), table)
    return out.reshape(position_ids.shape + (EMB,))
